# baseline (speedup 1.0000x reference)
import jax
import jax.numpy as jnp
from jax.experimental import pallas as pl

NBLOCKS = 4


def _dot(a, b):
    return jax.lax.dot_general(a, b, (((1,), (0,)), ((), ())),
                               preferred_element_type=jnp.float32)


def _apply(A, cur):
    nsplit = 4
    blk = A.shape[0] // nsplit
    parts = [_dot(A[i * blk:(i + 1) * blk, :], cur) for i in range(nsplit)]
    return jnp.concatenate(parts, axis=0)


def _krylov_body(adj_ref, feat_ref, w0_ref, b0_ref, w1_ref, b1_ref,
                 w2_ref, b2_ref, wout_ref, bout_ref, out_ref):
    A = adj_ref[...]
    nfeat = feat_ref.shape[1]
    nhid = w0_ref.shape[1]

    curs = [feat_ref[...]]
    for k in range(1, NBLOCKS):
        curs.append(_apply(A, curs[-1]))
    acc = b0_ref[...]
    for k in range(NBLOCKS):
        acc = acc + _dot(curs[k], w0_ref[k * nfeat:(k + 1) * nfeat, :])
    h = jnp.tanh(acc)

    for w_ref, b_ref in ((w1_ref, b1_ref), (w2_ref, b2_ref)):
        curs = [h]
        for k in range(1, NBLOCKS):
            curs.append(_apply(A, curs[-1]))
        acc = b_ref[...]
        for k in range(NBLOCKS):
            acc = acc + _dot(curs[k], w_ref[k * nhid:(k + 1) * nhid, :])
        h = jnp.tanh(acc)

    o = _dot(h, wout_ref[...]) + bout_ref[...]
    nrm = jnp.sqrt(jnp.sum(o * o, axis=1, keepdims=True))
    out_ref[...] = o / jnp.maximum(nrm, 1e-12)


def kernel(x, adj, features, W0, b0, W1, b1, W2, b2, Wout, bout):
    n = adj.shape[0]
    nclass = Wout.shape[1]
    return pl.pallas_call(
        _krylov_body,
        out_shape=jax.ShapeDtypeStruct((n, nclass), jnp.float32),
    )(adj, features, W0, b0.reshape(1, -1), W1, b1.reshape(1, -1),
      W2, b2.reshape(1, -1), Wout, bout.reshape(1, -1))


# speedup vs baseline: 1.1061x; 1.1061x over previous
import jax
import jax.numpy as jnp
from jax.experimental import pallas as pl

NBLOCKS = 4


def _dot(a, b):
    return jax.lax.dot_general(a, b, (((1,), (0,)), ((), ())),
                               preferred_element_type=jnp.float32)


def _wdot(m, w):
    half = m.shape[0] // 2
    return jnp.concatenate([_dot(m[:half, :], w), _dot(m[half:, :], w)], axis=0)


def _apply(A, cur):
    half = A.shape[0] // 2
    top = _dot(A[:half, :], cur)
    bot = _dot(A[half:, :], cur)
    return jnp.concatenate([top, bot], axis=0)


def _krylov_body(adj_ref, feat_ref, w0_ref, b0_ref, w1_ref, b1_ref,
                 w2_ref, b2_ref, wout_ref, bout_ref, out_ref):
    A = adj_ref[...]
    nfeat = feat_ref.shape[1]
    nhid = w0_ref.shape[1]

    curs = [feat_ref[...]]
    for k in range(1, NBLOCKS):
        curs.append(_apply(A, curs[-1]))
    acc = b0_ref[...]
    for k in range(NBLOCKS):
        acc = acc + _wdot(curs[k], w0_ref[k * nfeat:(k + 1) * nfeat, :])
    h = jnp.tanh(acc)

    for w_ref, b_ref in ((w1_ref, b1_ref), (w2_ref, b2_ref)):
        curs = [h]
        for k in range(1, NBLOCKS):
            curs.append(_apply(A, curs[-1]))
        acc = b_ref[...]
        for k in range(NBLOCKS):
            acc = acc + _wdot(curs[k], w_ref[k * nhid:(k + 1) * nhid, :])
        h = jnp.tanh(acc)

    o = _dot(h, wout_ref[...]) + bout_ref[...]
    nrm = jnp.sqrt(jnp.sum(o * o, axis=1, keepdims=True))
    out_ref[...] = o / jnp.maximum(nrm, 1e-12)


def kernel(x, adj, features, W0, b0, W1, b1, W2, b2, Wout, bout):
    n = adj.shape[0]
    nclass = Wout.shape[1]
    return pl.pallas_call(
        _krylov_body,
        out_shape=jax.ShapeDtypeStruct((n, nclass), jnp.float32),
    )(adj, features, W0, b0.reshape(1, -1), W1, b1.reshape(1, -1),
      W2, b2.reshape(1, -1), Wout, bout.reshape(1, -1))
